# Initial kernel scaffold; baseline (speedup 1.0000x reference)
import jax
import jax.numpy as jnp
from jax.experimental import pallas as pl


def kernel(table_idx, meta_table, embed_table, W_meta, b_meta, W_embed, b_embed, ln_g, ln_b, W_final, b_final):
    raise NotImplementedError("write your pallas kernel here")


# speedup vs baseline: 1.5639x; 1.5639x over previous
import functools

import jax
import jax.numpy as jnp
from jax import lax
from jax.experimental import pallas as pl
from jax.experimental.pallas import tpu as pltpu
from jax.experimental.pallas import tpu_sc as plsc

B = 16384
META_IN, META_OUT = 16, 32
EMB_IN, EMB_OUT = 128, 64
FINAL_IN = META_OUT + EMB_OUT
SIZE = 64

NC, NS = 2, 16
NW = NC * NS
BPW = B // NW
CHUNK = 128
NCHUNK = BPW // CHUNK

@functools.cache
def _make_sc_gather():
    mesh = plsc.VectorSubcoreMesh(core_axis_name="c", subcore_axis_name="s")

    @functools.partial(
        pl.kernel,
        out_type=(
            jax.ShapeDtypeStruct((B, META_IN), jnp.float32),
            jax.ShapeDtypeStruct((B, EMB_IN), jnp.float32),
        ),
        mesh=mesh,
        scratch_types=[
            pltpu.VMEM((NCHUNK, CHUNK), jnp.int32),
            pltpu.VMEM((BPW, META_IN), jnp.float32),
            pltpu.VMEM((BPW, EMB_IN), jnp.float32),
            pltpu.SemaphoreType.DMA,
        ],
        compiler_params=pltpu.CompilerParams(use_tc_tiling_on_sc=False),
    )
    def _sc_gather(idx_hbm, meta_hbm, embed_hbm, meta_out, embed_out,
                   idx_v, meta_v, emb_v, sem):
        wid = lax.axis_index("s") * NC + lax.axis_index("c")
        base = wid * BPW
        pltpu.sync_copy(idx_hbm.at[wid], idx_v)
        copies = []
        for j in range(NCHUNK):
            copies.append(pltpu.async_copy(
                meta_hbm.at[idx_v.at[j]],
                meta_v.at[pl.ds(j * CHUNK, CHUNK)], sem))
            copies.append(pltpu.async_copy(
                embed_hbm.at[idx_v.at[j]],
                emb_v.at[pl.ds(j * CHUNK, CHUNK)], sem))
        for c in copies:
            c.wait()
        pltpu.sync_copy(meta_v, meta_out.at[pl.ds(base, BPW)])
        pltpu.sync_copy(emb_v, embed_out.at[pl.ds(base, BPW)])

    return _sc_gather


def _gelu(x):
    return 0.5 * x * (1.0 + lax.erf(x * 0.7071067811865476))


def _head_body(mf_ref, ef_ref, wm_ref, bm_ref, we_ref, be_ref,
               g_ref, bln_ref, wf_ref, bf_ref, o_ref):
    mf = mf_ref[...]
    ef = ef_ref[...]
    meta = _gelu(jnp.dot(mf, wm_ref[...], preferred_element_type=jnp.float32)
                 + bm_ref[...])
    emb = (jnp.dot(ef, we_ref[...], preferred_element_type=jnp.float32)
           + be_ref[...])
    s = jnp.sum(meta, axis=-1, keepdims=True) + jnp.sum(emb, axis=-1, keepdims=True)
    ss = (jnp.sum(meta * meta, axis=-1, keepdims=True)
          + jnp.sum(emb * emb, axis=-1, keepdims=True))
    mu = s * (1.0 / FINAL_IN)
    var = ss * (1.0 / FINAL_IN) - mu * mu
    inv = lax.rsqrt(var + 1e-5)
    g = g_ref[...]
    bln = bln_ref[...]
    meta_n = (meta - mu) * inv * g[:, :META_OUT] + bln[:, :META_OUT]
    emb_n = (emb - mu) * inv * g[:, META_OUT:] + bln[:, META_OUT:]
    h = (jnp.dot(meta_n, wf_ref[:META_OUT, :], preferred_element_type=jnp.float32)
         + jnp.dot(emb_n, wf_ref[META_OUT:, :], preferred_element_type=jnp.float32)
         + bf_ref[...])
    o_ref[...] = _gelu(h)


BM = 2048


def _head(meta_feat, emb_feat, W_meta, b_meta, W_embed, b_embed,
          ln_g, ln_b, W_final, b_final):
    full = lambda shape: pl.BlockSpec(shape, lambda i: (0,) * len(shape))
    return pl.pallas_call(
        _head_body,
        grid=(B // BM,),
        in_specs=[
            pl.BlockSpec((BM, META_IN), lambda i: (i, 0)),
            pl.BlockSpec((BM, EMB_IN), lambda i: (i, 0)),
            full((META_IN, META_OUT)),
            full((1, META_OUT)),
            full((EMB_IN, EMB_OUT)),
            full((1, EMB_OUT)),
            full((1, FINAL_IN)),
            full((1, FINAL_IN)),
            full((FINAL_IN, SIZE)),
            full((1, SIZE)),
        ],
        out_specs=pl.BlockSpec((BM, SIZE), lambda i: (i, 0)),
        out_shape=jax.ShapeDtypeStruct((B, SIZE), jnp.float32),
        compiler_params=pltpu.CompilerParams(
            dimension_semantics=("arbitrary",)),
    )(meta_feat, emb_feat, W_meta, b_meta.reshape(1, -1), W_embed,
      b_embed.reshape(1, -1), ln_g.reshape(1, -1), ln_b.reshape(1, -1),
      W_final, b_final.reshape(1, -1))


def kernel(table_idx, meta_table, embed_table, W_meta, b_meta,
           W_embed, b_embed, ln_g, ln_b, W_final, b_final):
    idx = table_idx.astype(jnp.int32).reshape(NW, NCHUNK, CHUNK)
    meta_feat, emb_feat = _make_sc_gather()(idx, meta_table, embed_table)
    return _head(meta_feat, emb_feat, W_meta, b_meta, W_embed, b_embed,
                 ln_g, ln_b, W_final, b_final)
